# Initial kernel scaffold; baseline (speedup 1.0000x reference)
import jax
import jax.numpy as jnp
from jax.experimental import pallas as pl


def kernel(node_list, edge_index, true_batch_size, W1, b1, W2, b2, W3, b3, bn1_w, bn1_b, bn2_w, bn2_b):
    raise NotImplementedError("write your pallas kernel here")


# speedup vs baseline: 160.8314x; 160.8314x over previous
import functools

import jax
import jax.numpy as jnp
from jax import lax
from jax.experimental import pallas as pl
from jax.experimental.pallas import tpu as pltpu
from jax.experimental.pallas import tpu_sc as plsc

N = 625
NC = 640
NR = 626
FLAT = NR * NC
E = 20000
EP = 20480
NCORES = 2
NTILES = 16
EPT = EP // (NCORES * NTILES)
IDX_ROWS = EPT // 128
CHUNK = FLAT // NTILES
D_EMB = 64
EPS = 1e-5


def _sc_body(src_hbm, dst_hbm, out_hbm, src_v, dst_v, idx_v, one_v, stage_v, acc):
    cid = lax.axis_index("c")
    tid = lax.axis_index("s")

    ebase = pl.multiple_of((cid * NTILES + tid) * EPT, 8)
    pltpu.sync_copy(src_hbm.at[pl.ds(ebase, EPT)], src_v)
    pltpu.sync_copy(dst_hbm.at[pl.ds(ebase, EPT)], dst_v)

    def _zero(i, _):
        stage_v[pl.ds(i * 16, 16)] = jnp.zeros((16,), jnp.float32)
        return 0
    lax.fori_loop(0, CHUNK // 16, _zero, 0)
    cbase = pl.multiple_of(tid * CHUNK, 8)
    pltpu.sync_copy(stage_v, acc.at[pl.ds(cbase, CHUNK)])

    for i in range(EPT // 16):
        r, c = divmod(i * 16, 128)
        d16 = dst_v[pl.ds(i * 16, 16)]
        s16 = src_v[pl.ds(i * 16, 16)]
        idx_v[r, pl.ds(c, 16)] = d16 * NC + s16
        one_v[r, pl.ds(c, 16)] = jnp.full((16,), 1.0, jnp.float32)

    plsc.subcore_barrier()

    for j in range(IDX_ROWS):
        pltpu.sync_copy(one_v.at[j], acc.at[idx_v.at[j]], add=True)

    plsc.subcore_barrier()

    obase = pl.multiple_of(cid * FLAT + tid * CHUNK, 8)
    pltpu.sync_copy(acc.at[pl.ds(cbase, CHUNK)], stage_v)
    pltpu.sync_copy(stage_v, out_hbm.at[pl.ds(obase, CHUNK)])


def _sc_build_adj(src_p, dst_p):
    mesh = plsc.VectorSubcoreMesh(core_axis_name="c", subcore_axis_name="s",
                                  num_cores=NCORES)
    f = functools.partial(
        pl.kernel,
        mesh=mesh,
        out_type=jax.ShapeDtypeStruct((NCORES * FLAT,), jnp.float32),
        scratch_types=[
            pltpu.VMEM((EPT,), jnp.int32),
            pltpu.VMEM((EPT,), jnp.int32),
            pltpu.VMEM((IDX_ROWS, 128), jnp.int32),
            pltpu.VMEM((IDX_ROWS, 128), jnp.float32),
            pltpu.VMEM((CHUNK,), jnp.float32),
            pltpu.VMEM_SHARED((FLAT,), jnp.float32),
        ],
    )(_sc_body)
    return f(src_p, dst_p)


def _dot(a, b):
    return lax.dot_general(a, b, (((1,), (0,)), ((), ())),
                           preferred_element_type=jnp.float32)


def _bn(h, w, b):
    mu = jnp.mean(h, axis=0)
    xc = h - mu[None, :]
    var = jnp.mean(xc * xc, axis=0)
    return xc * lax.rsqrt(var + EPS) * w[None, :] + b[None, :]


def _layer(x, W, bvec, A, dinv, bsz):
    y = _dot(x, W)
    outs = []
    for g in range(bsz):
        yg = y[g * N:(g + 1) * N] * dinv
        tg = _dot(A, yg) + yg
        outs.append(tg * dinv)
    return jnp.concatenate(outs, axis=0) + bvec[None, :]


def _tc_body(bsz, a_ref, x_ref, w1_ref, b1_ref, w2_ref, b2_ref, w3_ref, b3_ref,
             g1_ref, be1_ref, g2_ref, be2_ref, o_ref):
    a = a_ref[0] + a_ref[1]
    a = a[:N, :]
    deg = jnp.sum(a, axis=1, keepdims=True) + 1.0
    dinv = lax.rsqrt(deg)
    A = a[:, :N]
    x = x_ref[...]
    h = _layer(x, w1_ref[...], b1_ref[...], A, dinv, bsz)
    h = jnp.maximum(h, 0.0)
    h = _bn(h, g1_ref[...], be1_ref[...])
    h = _layer(h, w2_ref[...], b2_ref[...], A, dinv, bsz)
    h = jnp.maximum(h, 0.0)
    h = _bn(h, g2_ref[...], be2_ref[...])
    h = _layer(h, w3_ref[...], b3_ref[...], A, dinv, bsz)
    o_ref[...] = jnp.maximum(h, 0.0)


def _tc_forward(a_parts, x2, W1, b1, W2, b2, W3, b3, bn1_w, bn1_b, bn2_w, bn2_b,
                bsz):
    return pl.pallas_call(
        functools.partial(_tc_body, bsz),
        out_shape=jax.ShapeDtypeStruct((bsz * N, D_EMB), jnp.float32),
    )(a_parts, x2, W1, b1, W2, b2, W3, b3, bn1_w, bn1_b, bn2_w, bn2_b)


def kernel(node_list, edge_index, true_batch_size, W1, b1, W2, b2, W3, b3,
           bn1_w, bn1_b, bn2_w, bn2_b):
    bsz, n_per_graph, feat = node_list.shape
    src = edge_index[0]
    dst = edge_index[1]
    pad = EP - src.shape[0]
    src_p = jnp.concatenate([src, jnp.zeros((pad,), jnp.int32)])
    dst_p = jnp.concatenate([dst, jnp.full((pad,), N, jnp.int32)])

    a_parts = _sc_build_adj(src_p, dst_p).reshape(NCORES, NR, NC)

    x2 = node_list.reshape(bsz * n_per_graph, feat)
    h = _tc_forward(a_parts, x2, W1, b1, W2, b2, W3, b3,
                    bn1_w, bn1_b, bn2_w, bn2_b, bsz)
    zero_residual = (jnp.asarray(true_batch_size) - bsz).astype(h.dtype)
    return h.reshape(bsz, n_per_graph, D_EMB) + zero_residual


# speedup vs baseline: 200.3926x; 1.2460x over previous
import functools

import jax
import jax.numpy as jnp
from jax import lax
from jax.experimental import pallas as pl
from jax.experimental.pallas import tpu as pltpu
from jax.experimental.pallas import tpu_sc as plsc

N = 625
NC = 640
NR = 626
FLAT = NR * NC
E = 20000
EP = 20480
NCORES = 2
NTILES = 16
EPT = EP // (NCORES * NTILES)
IDX_ROWS = EPT // 128
CHUNK = FLAT // NTILES
D_EMB = 64
EPS = 1e-5


def _sc_body(src_hbm, dst_hbm, out_hbm, src_v, dst_v, idx_v, one_v, stage_v, acc):
    cid = lax.axis_index("c")
    tid = lax.axis_index("s")

    ebase = pl.multiple_of((cid * NTILES + tid) * EPT, 8)
    pltpu.sync_copy(src_hbm.at[pl.ds(ebase, EPT)], src_v)
    pltpu.sync_copy(dst_hbm.at[pl.ds(ebase, EPT)], dst_v)

    def _zero(i, _):
        stage_v[pl.ds(i * 16, 16)] = jnp.zeros((16,), jnp.float32)
        return 0
    lax.fori_loop(0, CHUNK // 16, _zero, 0)
    cbase = pl.multiple_of(tid * CHUNK, 8)
    pltpu.sync_copy(stage_v, acc.at[pl.ds(cbase, CHUNK)])

    for i in range(EPT // 16):
        r, c = divmod(i * 16, 128)
        d16 = dst_v[pl.ds(i * 16, 16)]
        s16 = src_v[pl.ds(i * 16, 16)]
        idx_v[r, pl.ds(c, 16)] = d16 * NC + s16
        one_v[r, pl.ds(c, 16)] = jnp.full((16,), 1.0, jnp.float32)

    plsc.subcore_barrier()

    for j in range(IDX_ROWS):
        pltpu.sync_copy(one_v.at[j], acc.at[idx_v.at[j]], add=True)

    plsc.subcore_barrier()

    obase = pl.multiple_of(cid * FLAT + tid * CHUNK, 8)
    pltpu.sync_copy(acc.at[pl.ds(cbase, CHUNK)], stage_v)
    pltpu.sync_copy(stage_v, out_hbm.at[pl.ds(obase, CHUNK)])


def _sc_build_adj(src_p, dst_p):
    mesh = plsc.VectorSubcoreMesh(core_axis_name="c", subcore_axis_name="s",
                                  num_cores=NCORES)
    f = functools.partial(
        pl.kernel,
        mesh=mesh,
        out_type=jax.ShapeDtypeStruct((NCORES * FLAT,), jnp.float32),
        scratch_types=[
            pltpu.VMEM((EPT,), jnp.int32),
            pltpu.VMEM((EPT,), jnp.int32),
            pltpu.VMEM((IDX_ROWS, 128), jnp.int32),
            pltpu.VMEM((IDX_ROWS, 128), jnp.float32),
            pltpu.VMEM((CHUNK,), jnp.float32),
            pltpu.VMEM_SHARED((FLAT,), jnp.float32),
        ],
    )(_sc_body)
    return f(src_p, dst_p)


def _dot(a, b):
    return lax.dot_general(a, b, (((1,), (0,)), ((), ())),
                           preferred_element_type=jnp.float32)


def _agg(A, y, dinv):
    ys = y * dinv
    return (_dot(A, ys) + ys) * dinv


def _tile(v, bsz):
    return jnp.concatenate([v] * bsz, axis=1)


def _fold(s, bsz):
    acc = lax.slice(s, (0, 0), (1, D_EMB))
    for g in range(1, bsz):
        acc = acc + lax.slice(s, (0, g * D_EMB), (1, (g + 1) * D_EMB))
    return acc


def _bn_nm(h, w, b, bsz):
    cnt = float(bsz * N)
    s = jnp.sum(h, axis=0, keepdims=True)
    s2 = jnp.sum(h * h, axis=0, keepdims=True)
    mu = _fold(s, bsz) / cnt
    m2 = _fold(s2, bsz) / cnt
    var = m2 - mu * mu
    scale = lax.rsqrt(var + EPS) * w[None, :]
    shift = b[None, :] - mu * scale
    return h * _tile(scale, bsz) + _tile(shift, bsz)


def _tc_body(bsz, a_ref, x_ref, w1_ref, b1_ref, w2_ref, b2_ref, w3_ref, b3_ref,
             g1_ref, be1_ref, g2_ref, be2_ref, o_ref):
    a = a_ref[0] + a_ref[1]
    a = a[:N, :]
    deg = jnp.sum(a, axis=1, keepdims=True) + 1.0
    dinv = lax.rsqrt(deg)
    A = a[:, :N]

    w1 = w1_ref[...]
    y = jnp.concatenate([_dot(x_ref[g], w1) for g in range(bsz)], axis=1)
    h = _agg(A, y, dinv) + _tile(b1_ref[...][None, :], bsz)
    h = jnp.maximum(h, 0.0)
    h = _bn_nm(h, g1_ref[...], be1_ref[...], bsz)

    w2 = w2_ref[...]
    y = jnp.concatenate(
        [_dot(h[:, g * D_EMB:(g + 1) * D_EMB], w2) for g in range(bsz)], axis=1)
    h = _agg(A, y, dinv) + _tile(b2_ref[...][None, :], bsz)
    h = jnp.maximum(h, 0.0)
    h = _bn_nm(h, g2_ref[...], be2_ref[...], bsz)

    w3 = w3_ref[...]
    y = jnp.concatenate(
        [_dot(h[:, g * D_EMB:(g + 1) * D_EMB], w3) for g in range(bsz)], axis=1)
    h = _agg(A, y, dinv) + _tile(b3_ref[...][None, :], bsz)
    h = jnp.maximum(h, 0.0)

    for g in range(bsz):
        o_ref[g] = h[:, g * D_EMB:(g + 1) * D_EMB]


def _tc_forward(a_parts, x3, W1, b1, W2, b2, W3, b3, bn1_w, bn1_b, bn2_w, bn2_b,
                bsz):
    return pl.pallas_call(
        functools.partial(_tc_body, bsz),
        out_shape=jax.ShapeDtypeStruct((bsz, N, D_EMB), jnp.float32),
    )(a_parts, x3, W1, b1, W2, b2, W3, b3, bn1_w, bn1_b, bn2_w, bn2_b)


def kernel(node_list, edge_index, true_batch_size, W1, b1, W2, b2, W3, b3,
           bn1_w, bn1_b, bn2_w, bn2_b):
    bsz, n_per_graph, feat = node_list.shape
    src = edge_index[0]
    dst = edge_index[1]
    pad = EP - src.shape[0]
    src_p = jnp.concatenate([src, jnp.zeros((pad,), jnp.int32)])
    dst_p = jnp.concatenate([dst, jnp.full((pad,), N, jnp.int32)])

    a_parts = _sc_build_adj(src_p, dst_p).reshape(NCORES, NR, NC)

    h = _tc_forward(a_parts, node_list, W1, b1, W2, b2, W3, b3,
                    bn1_w, bn1_b, bn2_w, bn2_b, bsz)
    zero_residual = (jnp.asarray(true_batch_size) - bsz).astype(h.dtype)
    return h + zero_residual


# speedup vs baseline: 250.0896x; 1.2480x over previous
import functools

import jax
import jax.numpy as jnp
from jax import lax
from jax.experimental import pallas as pl
from jax.experimental.pallas import tpu as pltpu
from jax.experimental.pallas import tpu_sc as plsc

N = 625
NC = 640
NR = 626
FLAT = NR * NC
E = 20000
EP = 20480
NCORES = 1
NTILES = 16
EPT = EP // (NCORES * NTILES)
IDX_ROWS = EPT // 128
CHUNK = FLAT // NTILES
ZB = 3200
D_EMB = 64
EPS = 1e-5


def _sc_body(src_hbm, dst_hbm, out_hbm, src_v, dst_v, idx_v, one_v, stage_v,
             zb_v, sem, acc):
    tid = lax.axis_index("s")
    ebase = pl.multiple_of(tid * EPT, 8)
    cbase = pl.multiple_of(tid * CHUNK, 8)

    cps = [pltpu.async_copy(src_hbm.at[pl.ds(ebase, EPT)], src_v, sem),
           pltpu.async_copy(dst_hbm.at[pl.ds(ebase, EPT)], dst_v, sem)]
    for m in range(ZB // 16):
        zb_v[pl.ds(m * 16, 16)] = jnp.zeros((16,), jnp.float32)
    off = 0
    while off < CHUNK:
        sz = min(ZB, CHUNK - off)
        cps.append(pltpu.async_copy(zb_v.at[pl.ds(0, sz)],
                                    acc.at[pl.ds(cbase + off, sz)], sem))
        off += sz
    for cp in cps:
        cp.wait()

    for i in range(EPT // 16):
        r, c = divmod(i * 16, 128)
        d16 = dst_v[pl.ds(i * 16, 16)]
        s16 = src_v[pl.ds(i * 16, 16)]
        idx_v[r, pl.ds(c, 16)] = d16 * NC + s16
        one_v[r, pl.ds(c, 16)] = jnp.full((16,), 1.0, jnp.float32)

    plsc.subcore_barrier()

    cps = [pltpu.async_copy(one_v.at[j], acc.at[idx_v.at[j]], sem, add=True)
           for j in range(IDX_ROWS)]
    for cp in cps:
        cp.wait()

    plsc.subcore_barrier()

    pltpu.sync_copy(acc.at[pl.ds(cbase, CHUNK)], stage_v)
    pltpu.sync_copy(stage_v, out_hbm.at[pl.ds(cbase, CHUNK)])


def _sc_build_adj(src_p, dst_p):
    mesh = plsc.VectorSubcoreMesh(core_axis_name="c", subcore_axis_name="s",
                                  num_cores=NCORES)
    f = functools.partial(
        pl.kernel,
        mesh=mesh,
        out_type=jax.ShapeDtypeStruct((FLAT,), jnp.float32),
        scratch_types=[
            pltpu.VMEM((EPT,), jnp.int32),
            pltpu.VMEM((EPT,), jnp.int32),
            pltpu.VMEM((IDX_ROWS, 128), jnp.int32),
            pltpu.VMEM((IDX_ROWS, 128), jnp.float32),
            pltpu.VMEM((CHUNK,), jnp.float32),
            pltpu.VMEM((ZB,), jnp.float32),
            pltpu.SemaphoreType.DMA,
            pltpu.VMEM_SHARED((FLAT,), jnp.float32),
        ],
    )(_sc_body)
    return f(src_p, dst_p)


def _dot(a, b):
    return lax.dot_general(a, b, (((1,), (0,)), ((), ())),
                           preferred_element_type=jnp.float32)


def _agg(A, y, dinv):
    ys = y * dinv
    return (_dot(A, ys) + ys) * dinv


def _tile(v, bsz):
    return jnp.concatenate([v] * bsz, axis=1)


def _fold(s, bsz):
    acc = lax.slice(s, (0, 0), (1, D_EMB))
    for g in range(1, bsz):
        acc = acc + lax.slice(s, (0, g * D_EMB), (1, (g + 1) * D_EMB))
    return acc


def _bn_nm(h, w, b, bsz):
    cnt = float(bsz * N)
    s = jnp.sum(h, axis=0, keepdims=True)
    s2 = jnp.sum(h * h, axis=0, keepdims=True)
    mu = _fold(s, bsz) / cnt
    m2 = _fold(s2, bsz) / cnt
    var = m2 - mu * mu
    scale = lax.rsqrt(var + EPS) * w[None, :]
    shift = b[None, :] - mu * scale
    return h * _tile(scale, bsz) + _tile(shift, bsz)


def _tc_body(bsz, a_ref, x_ref, tb_ref, w1_ref, b1_ref, w2_ref, b2_ref,
             w3_ref, b3_ref, g1_ref, be1_ref, g2_ref, be2_ref, o_ref):
    a = a_ref[:N, :]
    deg = jnp.sum(a, axis=1, keepdims=True) + 1.0
    dinv = lax.rsqrt(deg)
    A = a[:, :N]

    w1 = w1_ref[...]
    y = jnp.concatenate([_dot(x_ref[g], w1) for g in range(bsz)], axis=1)
    h = _agg(A, y, dinv) + _tile(b1_ref[...][None, :], bsz)
    h = jnp.maximum(h, 0.0)
    h = _bn_nm(h, g1_ref[...], be1_ref[...], bsz)

    w2 = w2_ref[...]
    y = jnp.concatenate(
        [_dot(h[:, g * D_EMB:(g + 1) * D_EMB], w2) for g in range(bsz)], axis=1)
    h = _agg(A, y, dinv) + _tile(b2_ref[...][None, :], bsz)
    h = jnp.maximum(h, 0.0)
    h = _bn_nm(h, g2_ref[...], be2_ref[...], bsz)

    w3 = w3_ref[...]
    y = jnp.concatenate(
        [_dot(h[:, g * D_EMB:(g + 1) * D_EMB], w3) for g in range(bsz)], axis=1)
    h = _agg(A, y, dinv) + _tile(b3_ref[...][None, :], bsz)
    h = jnp.maximum(h, 0.0)
    h = h + (tb_ref[0, 0] - bsz).astype(jnp.float32)

    for g in range(bsz):
        o_ref[g] = h[:, g * D_EMB:(g + 1) * D_EMB]


def _tc_forward(a_mat, x3, tb, W1, b1, W2, b2, W3, b3, bn1_w, bn1_b, bn2_w,
                bn2_b, bsz):
    return pl.pallas_call(
        functools.partial(_tc_body, bsz),
        out_shape=jax.ShapeDtypeStruct((bsz, N, D_EMB), jnp.float32),
    )(a_mat, x3, tb, W1, b1, W2, b2, W3, b3, bn1_w, bn1_b, bn2_w, bn2_b)


def kernel(node_list, edge_index, true_batch_size, W1, b1, W2, b2, W3, b3,
           bn1_w, bn1_b, bn2_w, bn2_b):
    bsz, n_per_graph, feat = node_list.shape
    src = edge_index[0]
    dst = edge_index[1]
    pad = EP - src.shape[0]
    src_p = jnp.concatenate([src, jnp.zeros((pad,), jnp.int32)])
    dst_p = jnp.concatenate([dst, jnp.full((pad,), N, jnp.int32)])

    a_mat = _sc_build_adj(src_p, dst_p).reshape(NR, NC)

    tb = jnp.asarray(true_batch_size, jnp.int32).reshape(1, 1)
    return _tc_forward(a_mat, node_list, tb, W1, b1, W2, b2, W3, b3,
                       bn1_w, bn1_b, bn2_w, bn2_b, bsz)
